# baseline (speedup 1.0000x reference)
import functools

import jax
import jax.numpy as jnp
from jax import lax
from jax.experimental import pallas as pl
from jax.experimental.pallas import tpu as pltpu
from jax.experimental.pallas import tpu_sc as plsc

N = 10000
NP = 10240
E = 320000
H = 128
L = 4
OUT = 128
G = 128
ATOM_V = 119
BOND_V = 6

NBN = NP // 256
BE = 512
NBE = E // BE

NSUB = 16
EPT = E // NSUB
CH = 80
NCH = EPT // CH

ACC_R = 8640
RA = 8576
BTR = 64
RB = NP - RA


def _enc_nodes_body(nf_ref, tab_ref, out_ref):
    nf = nf_ref[...]
    acc = jnp.zeros((256, H), jnp.float32)
    for i in range(9):
        oh = (nf[:, i][:, None]
              == lax.broadcasted_iota(jnp.int32, (256, ATOM_V), 1)
              ).astype(jnp.float32)
        acc = acc + jnp.dot(oh, tab_ref[i], preferred_element_type=jnp.float32)
    out_ref[...] = acc


def _enc_nodes(nf_pad, atom_tables):
    return pl.pallas_call(
        _enc_nodes_body,
        grid=(NBN,),
        in_specs=[
            pl.BlockSpec((256, 9), lambda i: (i, 0)),
            pl.BlockSpec((9, ATOM_V, H), lambda i: (0, 0, 0)),
        ],
        out_specs=pl.BlockSpec((256, H), lambda i: (i, 0)),
        out_shape=jax.ShapeDtypeStruct((NP, H), jnp.float32),
    )(nf_pad, atom_tables)


def _enc_edges_body(ef_ref, tab_ref, out_ref):
    ef = ef_ref[...]
    acc = jnp.zeros((BE, 64), jnp.float32)
    for i in range(3):
        oh = (ef[:, i][:, None]
              == lax.broadcasted_iota(jnp.int32, (BE, BOND_V), 1)
              ).astype(jnp.float32)
        acc = acc + jnp.dot(oh, tab_ref[0, i], preferred_element_type=jnp.float32)
    out_ref[...] = acc


def _enc_edges(edge_feat, bond_tables_r):
    return pl.pallas_call(
        _enc_edges_body,
        grid=(NBE, 2),
        in_specs=[
            pl.BlockSpec((BE, 3), lambda i, j: (i, 0)),
            pl.BlockSpec((1, 3, BOND_V, 64), lambda i, j: (j, 0, 0, 0)),
        ],
        out_specs=pl.BlockSpec((BE, 64), lambda i, j: (j * NBE + i, 0)),
        out_shape=jax.ShapeDtypeStruct((2 * E, 64), jnp.float32),
    )(edge_feat, bond_tables_r)


def _node_mm_body(hv_ref, wa_ref, ba_ref, wd_ref, bd_ref, wb_ref, bb_ref,
                  we_ref, be_ref, ah_ref, d_ref, eb_ref):
    hvb = hv_ref[...]
    ah_ref[...] = (jnp.dot(hvb, wa_ref[...], preferred_element_type=jnp.float32)
                   + ba_ref[...])
    d_ref[...] = (jnp.dot(hvb, wd_ref[...], preferred_element_type=jnp.float32)
                  + bd_ref[...])
    eh = (jnp.dot(hvb, we_ref[0], preferred_element_type=jnp.float32)
          + be_ref[0])
    bh = (jnp.dot(hvb, wb_ref[0], preferred_element_type=jnp.float32)
          + bb_ref[0])
    eb_ref[...] = jnp.concatenate([eh, bh], axis=1)


def _node_mm(hv, wa, ba, wd, bd, wb_r, bb_r, we_r, be_r):
    half_w = pl.BlockSpec((1, H, 64), lambda i, j: (j, 0, 0))
    half_b = pl.BlockSpec((1, 1, 64), lambda i, j: (j, 0, 0))
    full_w = pl.BlockSpec((H, H), lambda i, j: (0, 0))
    full_b = pl.BlockSpec((1, H), lambda i, j: (0, 0))
    return pl.pallas_call(
        _node_mm_body,
        grid=(NBN, 2),
        in_specs=[
            pl.BlockSpec((256, H), lambda i, j: (i, 0)),
            full_w, full_b, full_w, full_b,
            half_w, half_b, half_w, half_b,
        ],
        out_specs=[
            pl.BlockSpec((256, H), lambda i, j: (i, 0)),
            pl.BlockSpec((256, H), lambda i, j: (i, 0)),
            pl.BlockSpec((256, 128), lambda i, j: (j * NBN + i, 0)),
        ],
        out_shape=[
            jax.ShapeDtypeStruct((NP, H), jnp.float32),
            jax.ShapeDtypeStruct((NP, H), jnp.float32),
            jax.ShapeDtypeStruct((2 * NP, 128), jnp.float32),
        ],
    )(hv, wa, ba, wd, bd, wb_r, bb_r, we_r, be_r)


def _edge_mm_body(helo_ref, hehi_ref, wc_ref, bc_ref, ce_ref):
    heb = jnp.concatenate([helo_ref[...], hehi_ref[...]], axis=1)
    ce_ref[...] = (jnp.dot(heb, wc_ref[0], preferred_element_type=jnp.float32)
                   + bc_ref[0])


def _edge_mm(he_split, wc_r, bc_r):
    return pl.pallas_call(
        _edge_mm_body,
        grid=(NBE, 2),
        in_specs=[
            pl.BlockSpec((BE, 64), lambda i, j: (i, 0)),
            pl.BlockSpec((BE, 64), lambda i, j: (NBE + i, 0)),
            pl.BlockSpec((1, H, 64), lambda i, j: (j, 0, 0)),
            pl.BlockSpec((1, 1, 64), lambda i, j: (j, 0, 0)),
        ],
        out_specs=pl.BlockSpec((BE, 64), lambda i, j: (j * NBE + i, 0)),
        out_shape=jax.ShapeDtypeStruct((2 * E, 64), jnp.float32),
    )(he_split, he_split, wc_r, bc_r)


def _hupd_body(hv_ref, ah_ref, ndlo_ref, ndhi_ref, g_ref, b_ref, out_ref):
    ndlo = ndlo_ref[...]
    ndhi = ndhi_ref[...]
    num = jnp.concatenate([ndlo[:, :64], ndhi[:, :64]], axis=1)
    den = jnp.concatenate([ndlo[:, 64:], ndhi[:, 64:]], axis=1)
    h = ah_ref[...] + num / (den + 1e-6)
    h = g_ref[...] * h + b_ref[...]
    out_ref[...] = hv_ref[...] + jnp.maximum(h, 0.0)


def _h_update(hv, ah, nd_split, gh, bh):
    return pl.pallas_call(
        _hupd_body,
        grid=(NBN,),
        in_specs=[
            pl.BlockSpec((256, H), lambda i: (i, 0)),
            pl.BlockSpec((256, H), lambda i: (i, 0)),
            pl.BlockSpec((256, 128), lambda i: (i, 0)),
            pl.BlockSpec((256, 128), lambda i: (NBN + i, 0)),
            pl.BlockSpec((1, H), lambda i: (0, 0)),
            pl.BlockSpec((1, H), lambda i: (0, 0)),
        ],
        out_specs=pl.BlockSpec((256, H), lambda i: (i, 0)),
        out_shape=jax.ShapeDtypeStruct((NP, H), jnp.float32),
    )(hv, ah, nd_split, nd_split, gh, bh)


def _pool_body(hv_ref, gid_ref, wo_ref, bo_ref, out_ref, sums_ref, cnt_ref):
    i = pl.program_id(0)

    @pl.when(i == 0)
    def _():
        sums_ref[...] = jnp.zeros_like(sums_ref)
        cnt_ref[...] = jnp.zeros_like(cnt_ref)

    g = gid_ref[0, 0, :]
    oh = (g[:, None] == lax.broadcasted_iota(jnp.int32, (256, G), 1)
          ).astype(jnp.float32)
    dn = (((0,), (0,)), ((), ()))
    sums_ref[...] += lax.dot_general(oh, hv_ref[...], dn,
                                     preferred_element_type=jnp.float32)
    cnt_ref[...] += lax.dot_general(oh, jnp.ones((256, H), jnp.float32), dn,
                                    preferred_element_type=jnp.float32)

    @pl.when(i == NBN - 1)
    def _():
        hg = sums_ref[...] / jnp.maximum(cnt_ref[...], 1.0)
        out_ref[...] = (jnp.dot(hg, wo_ref[...],
                                preferred_element_type=jnp.float32)
                        + bo_ref[...])


def _pool(hv, gid_r, w_out, b_out):
    return pl.pallas_call(
        _pool_body,
        grid=(NBN,),
        in_specs=[
            pl.BlockSpec((256, H), lambda i: (i, 0)),
            pl.BlockSpec((1, 1, 256), lambda i: (i, 0, 0)),
            pl.BlockSpec((H, OUT), lambda i: (0, 0)),
            pl.BlockSpec((1, OUT), lambda i: (0, 0)),
        ],
        out_specs=pl.BlockSpec((G, OUT), lambda i: (0, 0)),
        out_shape=jax.ShapeDtypeStruct((G, OUT), jnp.float32),
        scratch_shapes=[
            pltpu.VMEM((G, H), jnp.float32),
            pltpu.VMEM((G, H), jnp.float32),
        ],
    )(hv, gid_r, w_out, b_out)


_SC_MESH = plsc.VectorSubcoreMesh(core_axis_name="c", subcore_axis_name="s",
                                  num_cores=1)


@functools.partial(
    pl.kernel,
    mesh=_SC_MESH,
    out_type=(
        jax.ShapeDtypeStruct((2 * E, 64), jnp.float32),
        jax.ShapeDtypeStruct((2 * NP, 128), jnp.float32),
        jax.ShapeDtypeStruct((2 * E, 128), jnp.float32),
    ),
    scratch_types=[
        pltpu.VMEM((CH,), jnp.int32),
        pltpu.VMEM((CH,), jnp.int32),
        pltpu.VMEM((CH, 128), jnp.float32),
        pltpu.VMEM((CH, 128), jnp.float32),
        pltpu.VMEM((CH, 64), jnp.float32),
        pltpu.VMEM((CH, 64), jnp.float32),
        pltpu.VMEM((CH, 128), jnp.float32),
        pltpu.VMEM((128,), jnp.float32),
        pltpu.VMEM((128,), jnp.float32),
        pltpu.VMEM((CH, 128), jnp.float32),
        pltpu.VMEM_SHARED((ACC_R, 128), jnp.float32),
        pltpu.SemaphoreType.DMA,
    ],
)
def _edge_sc(src_hbm, dst_hbm, ce_hbm, he_hbm, d_hbm, eb_hbm, ge_hbm, be_hbm,
             hen_hbm, nd_hbm, ms_hbm,
             dst_v, idxs_v, d_v, eb_v, ce_v, he_v, ms_v,
             g_v, b_v, z_v, nd_sh, sem):
    s = lax.axis_index("s")

    zero16 = jnp.zeros((16,), jnp.float32)

    def zbuf(i, carry):
        z_v[i // 8, pl.ds((i % 8) * 16, 16)] = zero16
        return carry

    lax.fori_loop(0, CH * 8, zbuf, 0)

    pltpu.sync_copy(ge_hbm, g_v)
    pltpu.sync_copy(be_hbm, b_v)
    base = s * EPT
    ra16 = jnp.full((16,), RA, jnp.int32)
    rb16 = jnp.full((16,), RA - BTR, jnp.int32)
    zero16i = jnp.zeros((16,), jnp.int32)

    def zero_rows(nrows, carry_tag):
        nchunks = nrows // 64
        per_tile = (nchunks + NSUB - 1) // NSUB

        def zacc(q, carry):
            idx = s * per_tile + q

            @pl.when(idx < nchunks)
            def _():
                pltpu.sync_copy(z_v.at[pl.ds(0, 64)],
                                nd_sh.at[pl.ds(idx * 64, 64)])
            return carry

        lax.fori_loop(0, per_tile, zacc, carry_tag)

    for ch in (0, 1):
        cnp = ch * NP
        c64 = ch * 64

        zero_rows(ACC_R, 0)
        plsc.subcore_barrier()

        gs = [g_v[pl.ds(c64 + f * 16, 16)] for f in range(4)]
        bs = [b_v[pl.ds(c64 + f * 16, 16)] for f in range(4)]
        cnp16 = jnp.full((16,), ch * NP, jnp.int32)

        def chunk(k, carry):
            e0 = base + k * CH
            pltpu.sync_copy(dst_hbm.at[pl.ds(e0, CH)], dst_v)
            pltpu.sync_copy(src_hbm.at[pl.ds(e0, CH)], idxs_v)

            def mkidx(gi, cc):
                sl = pl.ds(gi * 16, 16)
                idxs_v[sl] = idxs_v[sl] + cnp16
                return cc

            lax.fori_loop(0, CH // 16, mkidx, 0)

            pltpu.async_copy(d_hbm.at[dst_v], d_v, sem).wait()
            pltpu.async_copy(eb_hbm.at[idxs_v], eb_v, sem).wait()
            pltpu.sync_copy(ce_hbm.at[pl.ds(ch * E + e0, CH)], ce_v)
            pltpu.sync_copy(he_hbm.at[pl.ds(ch * E + e0, CH)], he_v)

            def edge(e, cc):
                for f in range(4):
                    sl = pl.ds(f * 16, 16)
                    en = (d_v[e, pl.ds(c64 + f * 16, 16)] + eb_v[e, sl]
                          + ce_v[e, sl])
                    sg = 1.0 / (1.0 + jnp.exp(-en))
                    ms_v[e, sl] = sg * eb_v[e, pl.ds(64 + f * 16, 16)]
                    ms_v[e, pl.ds(64 + f * 16, 16)] = sg
                    he_v[e, sl] = he_v[e, sl] + jnp.maximum(
                        gs[f] * en + bs[f], 0.0)
                return cc

            lax.fori_loop(0, CH, edge, 0)

            pltpu.sync_copy(he_v, hen_hbm.at[pl.ds(ch * E + e0, CH)])
            pltpu.sync_copy(ms_v, ms_hbm.at[pl.ds(ch * E + e0, CH)])

            def clampa(gi, cc):
                sl = pl.ds(gi * 16, 16)
                idxs_v[sl] = jnp.minimum(dst_v[sl], ra16)
                return cc

            lax.fori_loop(0, CH // 16, clampa, 0)
            pltpu.sync_copy(ms_v, nd_sh.at[idxs_v], add=True)
            return carry

        lax.fori_loop(0, NCH, chunk, 0)
        plsc.subcore_barrier()

        npa = (RA // 64 + NSUB - 1) // NSUB

        def dump_a(q, carry):
            idx = s * npa + q

            @pl.when(idx < RA // 64)
            def _():
                r0 = idx * 64
                pltpu.sync_copy(nd_sh.at[pl.ds(r0, 64)], ms_v.at[pl.ds(0, 64)])
                pltpu.sync_copy(ms_v.at[pl.ds(0, 64)],
                                nd_hbm.at[pl.ds(cnp + r0, 64)])
            return carry

        lax.fori_loop(0, npa, dump_a, 0)
        plsc.subcore_barrier()

        zero_rows(BTR + RB, 0)
        plsc.subcore_barrier()

        def chunk_b(k, carry):
            e0 = base + k * CH
            pltpu.sync_copy(dst_hbm.at[pl.ds(e0, CH)], dst_v)
            pltpu.sync_copy(ms_hbm.at[pl.ds(ch * E + e0, CH)], ms_v)

            def clampb(gi, cc):
                sl = pl.ds(gi * 16, 16)
                idxs_v[sl] = jnp.maximum(dst_v[sl] - rb16, zero16i)
                return cc

            lax.fori_loop(0, CH // 16, clampb, 0)
            pltpu.sync_copy(ms_v, nd_sh.at[idxs_v], add=True)
            return carry

        lax.fori_loop(0, NCH, chunk_b, 0)
        plsc.subcore_barrier()

        npb = (RB // 64 + NSUB - 1) // NSUB

        def dump_b(q, carry):
            idx = s * npb + q

            @pl.when(idx < RB // 64)
            def _():
                r0 = idx * 64
                pltpu.sync_copy(nd_sh.at[pl.ds(BTR + r0, 64)],
                                ms_v.at[pl.ds(0, 64)])
                pltpu.sync_copy(ms_v.at[pl.ds(0, 64)],
                                nd_hbm.at[pl.ds(cnp + RA + r0, 64)])
            return carry

        lax.fori_loop(0, npb, dump_b, 0)
        plsc.subcore_barrier()


def _split_w(w):
    return w.reshape(H, 2, 64).swapaxes(0, 1)


def _split_b(b):
    return b.reshape(2, 1, 64)


def kernel(node_feat, edge_feat, edge_index, node_graph_ids, atom_tables,
           bond_tables, WA, bA, WB, bB, WC, bC, WD, bD, WE, bE,
           gamma_h, beta_h, gamma_e, beta_e, W_out, b_out):
    nf_pad = jnp.pad(node_feat, ((0, NP - N), (0, 0)))
    gid_pad = jnp.pad(node_graph_ids, (0, NP - N), constant_values=G)
    gid_r = gid_pad.reshape(NBN, 1, 256)
    src = edge_index[0]
    dst = edge_index[1]
    bond_r = bond_tables.reshape(3, BOND_V, 2, 64).transpose(2, 0, 1, 3)

    hv = _enc_nodes(nf_pad, atom_tables)
    he = _enc_edges(edge_feat, bond_r)

    for l in range(L):
        ah, d_t, eb_t = _node_mm(
            hv, WA[l], bA[l].reshape(1, H),
            WD[l], bD[l].reshape(1, H),
            _split_w(WB[l]), _split_b(bB[l]),
            _split_w(WE[l]), _split_b(bE[l]))
        ce = _edge_mm(he, _split_w(WC[l]), _split_b(bC[l]))
        he, nd, _ = _edge_sc(src, dst, ce, he, d_t, eb_t,
                             gamma_e[l], beta_e[l])
        hv = _h_update(hv, ah, nd,
                       gamma_h[l].reshape(1, H), beta_h[l].reshape(1, H))

    return _pool(hv, gid_r, W_out, b_out.reshape(1, OUT))
